# baseline (speedup 1.0000x reference)
import functools

import jax
import jax.numpy as jnp
from jax.experimental import pallas as pl
from jax.experimental.pallas import tpu as pltpu


def _matmul_kernel(x_ref, w_ref, b_ref, o_ref):
    o_ref[...] = (
        jnp.dot(x_ref[...], w_ref[...], preferred_element_type=jnp.float32)
        + b_ref[...]
    )


def _matmul(x, wT, b, block_rows=1000):
    n, k = x.shape
    m = wT.shape[1]
    grid = (n // block_rows,)
    return pl.pallas_call(
        _matmul_kernel,
        grid=grid,
        in_specs=[
            pl.BlockSpec((block_rows, k), lambda i: (i, 0)),
            pl.BlockSpec((k, m), lambda i: (0, 0)),
            pl.BlockSpec((1, m), lambda i: (0, 0)),
        ],
        out_specs=pl.BlockSpec((block_rows, m), lambda i: (i, 0)),
        out_shape=jax.ShapeDtypeStruct((n, m), jnp.float32),
    )(x, wT, b)


def _gatv2_layer(x, src, dst, ea, Wl, bl, Wr, br, We, att, bias):
    n = x.shape[0]
    xl = _matmul(x, Wl.T, bl[None, :])
    xr = _matmul(x, Wr.T, br[None, :])
    ee = ea @ We.T
    xr_pad = jnp.concatenate([xr, jnp.zeros((1, xr.shape[1]), dtype=xr.dtype)], axis=0)
    m = xl[src] + xr_pad[dst] + ee
    m = jnp.where(m > 0.0, m, 0.2 * m)
    e = jnp.sum(m * att, axis=-1)
    emax = jax.ops.segment_max(e, dst, num_segments=n + 1)
    ex = jnp.exp(e - emax[dst])
    den = jax.ops.segment_sum(ex, dst, num_segments=n + 1)
    alpha = ex / (den[dst] + 1e-16)
    return jax.ops.segment_sum(xl[src] * alpha[:, None], dst, num_segments=n + 1)[:n] + bias


def kernel(x, edge_index, edge_attr, Wl1, bl1, Wr1, br1, We1, att1, bias1,
           Wl2, bl2, Wr2, br2, We2, att2, bias2, Wc, bc):
    n = x.shape[0]
    src0 = edge_index[0]
    dst0 = edge_index[1]
    keep = src0 != dst0
    loop = jnp.arange(n)
    src = jnp.concatenate([src0, loop])
    dst = jnp.concatenate([jnp.where(keep, dst0, n), loop])
    ea = edge_attr[:, None]
    cnt = jnp.sum(keep).astype(ea.dtype)
    mean_ea = jnp.sum(ea * keep[:, None].astype(ea.dtype), axis=0, keepdims=True) / cnt
    ea_full = jnp.concatenate([ea, jnp.broadcast_to(mean_ea, (n, ea.shape[1]))], axis=0)
    h = _gatv2_layer(x, src, dst, ea_full, Wl1, bl1, Wr1, br1, We1, att1, bias1)
    h = jax.nn.elu(h)
    h = _gatv2_layer(h, src, dst, ea_full, Wl2, bl2, Wr2, br2, We2, att2, bias2)
    h = jax.nn.elu(h)
    pooled = jnp.mean(h, axis=0, keepdims=True)
    return pooled @ Wc.T + bc


# speedup vs baseline: 2.1853x; 2.1853x over previous
import jax
import jax.numpy as jnp
from jax import lax
from jax.experimental import pallas as pl
from jax.experimental.pallas import tpu as pltpu
from jax.experimental.pallas import tpu_sc as plsc

N_NODES = 10000
D_H = 512
NQ = 4
QW = D_H // NQ
NC, NS, LANES = 2, 16, 16
NW = NC * NS
EPT = 5376
EP = NW * EPT
KA = 16
GA = EPT // KA
NROWS = 10112
RPT = NROWS // NS
DW = 8


def _mm_kernel(x_ref, w_ref, b_ref, o_ref):
    o_ref[...] = (
        jnp.dot(x_ref[...], w_ref[...], preferred_element_type=jnp.float32)
        + b_ref[...]
    )


def _matmul(x, wT, b, block_rows=1000):
    n, k = x.shape
    m = wT.shape[1]
    return pl.pallas_call(
        _mm_kernel,
        grid=(n // block_rows,),
        in_specs=[
            pl.BlockSpec((block_rows, k), lambda i: (i, 0)),
            pl.BlockSpec((k, m), lambda i: (0, 0)),
            pl.BlockSpec((1, m), lambda i: (0, 0)),
        ],
        out_specs=pl.BlockSpec((block_rows, m), lambda i: (i, 0)),
        out_shape=jax.ShapeDtypeStruct((n, m), jnp.float32),
    )(x, wT, b[None, :])


def _mean_ea_kernel(ei_ref, ea_ref, o_ref):
    s_ids = ei_ref[0]
    d_ids = ei_ref[1]
    keep = s_ids != d_ids
    cnt = jnp.sum(keep.astype(jnp.float32))
    ssum = jnp.sum(jnp.where(keep, ea_ref[...], 0.0))
    o_ref[...] = jnp.full((8, 128), ssum / cnt, dtype=jnp.float32)


def _mean_ea(edge_index, edge_attr):
    e = edge_attr.shape[0]
    rows = e // 128
    out = pl.pallas_call(
        _mean_ea_kernel,
        out_shape=jax.ShapeDtypeStruct((8, 128), jnp.float32),
    )(edge_index.reshape(2, rows, 128), edge_attr.reshape(rows, 128))
    return out[0, 0]


def _sc_edge_body(xl, xr, q0, q1, q2, q3, srcg, dstg2, eag, wvec,
                  attvec, zn, zd, nump, denp,
                  num_sh, den_sh, src_v, didx, dst2_v, ea_v, ex_v,
                  rbuf, qbuf, dbuf, w_v, att_v):
    c = lax.axis_index("c")
    s = lax.axis_index("s")
    wid = c * NS + s
    qtabs = (q0, q1, q2, q3)

    pltpu.sync_copy(srcg.at[wid], src_v)
    pltpu.sync_copy(dstg2.at[wid], dst2_v)
    pltpu.sync_copy(eag.at[wid], ea_v)
    pltpu.sync_copy(wvec, w_v)
    pltpu.sync_copy(attvec, att_v)
    pltpu.sync_copy(zd.at[pl.ds(0, KA)], dbuf)

    lanes_i = lax.iota(jnp.int32, LANES)
    nclamp = jnp.full((LANES,), N_NODES - 1, jnp.int32)

    def body_a(g, _):
        base = g * KA
        for half in range(KA // LANES):
            dch = dst2_v[g, pl.ds(half * LANES, LANES)]
            didx[pl.ds(half * LANES, LANES)] = jnp.minimum(dch, nclamp)
        pltpu.sync_copy(xl.at[src_v.at[pl.ds(base, KA)]], rbuf)
        pltpu.sync_copy(xr.at[didx], rbuf, add=True)

        def sub(j, _):
            j16 = j * LANES
            eav = ea_v[pl.ds(base + j16, LANES)]
            eaus = [eav[u] for u in range(LANES)]
            accs = [jnp.zeros((LANES,), jnp.float32) for _ in range(LANES)]
            for cidx in range(D_H // LANES):
                wc = w_v[pl.ds(cidx * LANES, LANES)]
                ac = att_v[pl.ds(cidx * LANES, LANES)]
                for u in range(LANES):
                    row = rbuf[j16 + u, pl.ds(cidx * LANES, LANES)]
                    t = row + eaus[u] * wc
                    t = jnp.maximum(t, 0.2 * t)
                    accs[u] = accs[u] + t * ac
            e16 = jnp.zeros((LANES,), jnp.float32)
            for u in range(LANES):
                e16 = jnp.where(lanes_i == u, jnp.sum(accs[u]), e16)
            ex_v[pl.ds(base + j16, LANES)] = jnp.exp(
                jnp.minimum(e16, 50.0))
            return 0

        lax.fori_loop(0, KA // LANES, sub, 0)
        return 0

    lax.fori_loop(0, GA, body_a, 0)

    for q in range(NQ):
        pltpu.sync_copy(zn, num_sh.at[pl.ds(s * RPT, RPT)])
        if q == 0:
            pltpu.sync_copy(zd.at[pl.ds(0, RPT)],
                            den_sh.at[pl.ds(s * RPT, RPT)])
        plsc.subcore_barrier()

        def body_b(g, _):
            base = g * KA
            pltpu.sync_copy(qtabs[q].at[src_v.at[pl.ds(base, KA)]], qbuf)

            def sc(j, _):
                j16 = j * LANES
                exch = ex_v[pl.ds(base + j16, LANES)]
                for u in range(LANES):
                    exj = exch[u]
                    for cc in range(QW // LANES):
                        sl = pl.ds(cc * LANES, LANES)
                        qbuf[j16 + u, sl] = qbuf[j16 + u, sl] * exj
                return 0

            lax.fori_loop(0, KA // LANES, sc, 0)
            pltpu.sync_copy(qbuf, num_sh.at[dst2_v.at[g]], add=True)
            if q == 0:
                zeros_i = jnp.zeros((LANES,), jnp.int32)
                for subg in range(KA // LANES):
                    exv = ex_v[pl.ds(base + subg * LANES, LANES)]
                    plsc.store_scatter(
                        dbuf, [lanes_i + subg * LANES, zeros_i], exv)
                pltpu.sync_copy(dbuf, den_sh.at[dst2_v.at[g]], add=True)
            return 0

        lax.fori_loop(0, GA, body_b, 0)
        plsc.subcore_barrier()
        pltpu.sync_copy(num_sh.at[pl.ds(s * RPT, RPT)],
                        nump.at[c, q, pl.ds(s * RPT, RPT)])
        if q == 0:
            pltpu.sync_copy(den_sh.at[pl.ds(s * RPT, RPT)],
                            denp.at[c, pl.ds(s * RPT, RPT)])
        plsc.subcore_barrier()


def _sc_edge(xl, xr, quarters, srcg, dstg2, eag, wvec, attvec, zn, zd):
    mesh = plsc.VectorSubcoreMesh(core_axis_name="c", subcore_axis_name="s",
                                  num_cores=NC, num_subcores=NS)
    run = pl.kernel(
        _sc_edge_body,
        out_type=[
            jax.ShapeDtypeStruct((NC, NQ, NROWS, QW), jnp.float32),
            jax.ShapeDtypeStruct((NC, NROWS, DW), jnp.float32),
        ],
        mesh=mesh,
        compiler_params=pltpu.CompilerParams(needs_layout_passes=False,
                                             use_tc_tiling_on_sc=False),
        scratch_types=[
            pltpu.VMEM_SHARED((NROWS, QW), jnp.float32),
            pltpu.VMEM_SHARED((NROWS, DW), jnp.float32),
            pltpu.VMEM((EPT,), jnp.int32),
            pltpu.VMEM((KA,), jnp.int32),
            pltpu.VMEM((GA, KA), jnp.int32),
            pltpu.VMEM((EPT,), jnp.float32),
            pltpu.VMEM((EPT,), jnp.float32),
            pltpu.VMEM((KA, D_H), jnp.float32),
            pltpu.VMEM((KA, QW), jnp.float32),
            pltpu.VMEM((KA, DW), jnp.float32),
            pltpu.VMEM((D_H,), jnp.float32),
            pltpu.VMEM((D_H,), jnp.float32),
        ],
    )
    return run(xl, xr, quarters[0], quarters[1], quarters[2], quarters[3],
               srcg, dstg2, eag, wvec, attvec, zn, zd)


def _div_kernel(nmp_ref, dnp_ref, b_ref, o_ref):
    num = nmp_ref[...]
    den = dnp_ref[...]
    ns = num[0] + num[1]
    d = den[0, :, 0] + den[1, :, 0]
    h = jnp.concatenate([ns[0], ns[1], ns[2], ns[3]], axis=1)
    h = h / (d[:, None] + 1e-16) + b_ref[...]
    o_ref[...] = jnp.where(h > 0, h, jnp.exp(h) - 1.0)


def _divide_elu(nump, denp, bias, block_rows=1000):
    return pl.pallas_call(
        _div_kernel,
        grid=(N_NODES // block_rows,),
        in_specs=[
            pl.BlockSpec((NC, NQ, block_rows, QW), lambda i: (0, 0, i, 0)),
            pl.BlockSpec((NC, block_rows, DW), lambda i: (0, i, 0)),
            pl.BlockSpec((1, D_H), lambda i: (0, 0)),
        ],
        out_specs=pl.BlockSpec((block_rows, D_H), lambda i: (i, 0)),
        out_shape=jax.ShapeDtypeStruct((N_NODES, D_H), jnp.float32),
    )(nump, denp, bias[None, :])


def _final_kernel(nmp_ref, dnp_ref, b_ref, wc_ref, bc_ref, o_ref, acc_ref):
    i = pl.program_id(0)
    num = nmp_ref[...]
    den = dnp_ref[...]
    ns = num[0] + num[1]
    d = den[0, :, 0] + den[1, :, 0]
    h = jnp.concatenate([ns[0], ns[1], ns[2], ns[3]], axis=1)
    h = h / (d[:, None] + 1e-16) + b_ref[...]
    h = jnp.where(h > 0, h, jnp.exp(h) - 1.0)
    part = jnp.sum(h.reshape(-1, 8, D_H), axis=0)

    @pl.when(i == 0)
    def _():
        acc_ref[...] = part

    @pl.when(i > 0)
    def _():
        acc_ref[...] = acc_ref[...] + part

    @pl.when(i == pl.num_programs(0) - 1)
    def _():
        pooled = jnp.sum(acc_ref[...], axis=0, keepdims=True) / N_NODES
        res = jnp.dot(pooled, wc_ref[...],
                      preferred_element_type=jnp.float32) + bc_ref[...]
        o_ref[...] = jnp.broadcast_to(res, (8, 128))


def _final(nump, denp, bias, wcTp, bcp, block_rows=1000):
    return pl.pallas_call(
        _final_kernel,
        grid=(N_NODES // block_rows,),
        in_specs=[
            pl.BlockSpec((NC, NQ, block_rows, QW), lambda i: (0, 0, i, 0)),
            pl.BlockSpec((NC, block_rows, DW), lambda i: (0, i, 0)),
            pl.BlockSpec((1, D_H), lambda i: (0, 0)),
            pl.BlockSpec((D_H, 128), lambda i: (0, 0)),
            pl.BlockSpec((1, 128), lambda i: (0, 0)),
        ],
        out_specs=pl.BlockSpec((8, 128), lambda i: (0, 0)),
        out_shape=jax.ShapeDtypeStruct((8, 128), jnp.float32),
        scratch_shapes=[pltpu.VMEM((8, D_H), jnp.float32)],
    )(nump, denp, bias[None, :], wcTp, bcp)


def kernel(x, edge_index, edge_attr, Wl1, bl1, Wr1, br1, We1, att1, bias1,
           Wl2, bl2, Wr2, br2, We2, att2, bias2, Wc, bc):
    n = x.shape[0]
    e = edge_attr.shape[0]
    src0 = edge_index[0].astype(jnp.int32)
    dst0 = edge_index[1].astype(jnp.int32)
    keep = src0 != dst0
    mean_ea = _mean_ea(edge_index.astype(jnp.int32), edge_attr)
    loop = jnp.arange(n, dtype=jnp.int32)
    pad = EP - e - n
    src_all = jnp.concatenate([src0, loop, jnp.zeros((pad,), jnp.int32)])
    dst_all = jnp.concatenate([
        jnp.where(keep, dst0, n), loop, jnp.full((pad,), n, jnp.int32)])
    ea_all = jnp.concatenate([
        edge_attr, jnp.full((n,), mean_ea, jnp.float32),
        jnp.zeros((pad,), jnp.float32)])

    srcg = src_all.reshape(NW, EPT)
    dstg2 = dst_all.reshape(NW, GA, KA)
    eag = ea_all.reshape(NW, EPT)
    zn = jnp.zeros((RPT, QW), jnp.float32)
    zd = jnp.zeros((RPT, DW), jnp.float32)

    def layer(xin, Wl, bl, Wr, br, We, att):
        wT = jnp.concatenate([Wl, Wr], axis=0).T
        bcat = jnp.concatenate([bl, br])
        y = _matmul(xin, wT, bcat)
        xlv = y[:, :D_H]
        xrv = y[:, D_H:]
        quarters = [xlv[:, i * QW:(i + 1) * QW] for i in range(NQ)]
        return _sc_edge(xlv, xrv, quarters, srcg, dstg2, eag,
                        We[:, 0], att, zn, zd)

    nump1, denp1 = layer(x, Wl1, bl1, Wr1, br1, We1, att1)
    h = _divide_elu(nump1, denp1, bias1)
    nump2, denp2 = layer(h, Wl2, bl2, Wr2, br2, We2, att2)
    wcTp = jnp.pad(Wc.T, ((0, 0), (0, 128 - Wc.shape[0])))
    bcp = jnp.pad(bc, (0, 128 - bc.shape[0]))[None, :]
    outp = _final(nump2, denp2, bias2, wcTp, bcp)
    return outp[0:1, 0:Wc.shape[0]]


# speedup vs baseline: 3.0558x; 1.3984x over previous
import jax
import jax.numpy as jnp
from jax import lax
from jax.experimental import pallas as pl
from jax.experimental.pallas import tpu as pltpu
from jax.experimental.pallas import tpu_sc as plsc

N_NODES = 10000
D_H = 512
NQ = 4
QW = D_H // NQ
NC, NS, LANES = 2, 16, 16
NW = NC * NS
EPT = 5376
EP = NW * EPT
KA = 16
GA = EPT // KA
NROWS = 10112
RPT = NROWS // NS
DW = 8


def _mm_kernel(x_ref, w_ref, b_ref, o_ref):
    o_ref[...] = (
        jnp.dot(x_ref[...], w_ref[...], preferred_element_type=jnp.float32)
        + b_ref[...]
    )


def _matmul(x, wT, b, block_rows=1000):
    n, k = x.shape
    m = wT.shape[1]
    return pl.pallas_call(
        _mm_kernel,
        grid=(n // block_rows,),
        in_specs=[
            pl.BlockSpec((block_rows, k), lambda i: (i, 0)),
            pl.BlockSpec((k, m), lambda i: (0, 0)),
            pl.BlockSpec((1, m), lambda i: (0, 0)),
        ],
        out_specs=pl.BlockSpec((block_rows, m), lambda i: (i, 0)),
        out_shape=jax.ShapeDtypeStruct((n, m), jnp.float32),
    )(x, wT, b[None, :])


def _mean_ea_kernel(ei_ref, ea_ref, o_ref):
    s_ids = ei_ref[0]
    d_ids = ei_ref[1]
    keep = s_ids != d_ids
    cnt = jnp.sum(keep.astype(jnp.float32))
    ssum = jnp.sum(jnp.where(keep, ea_ref[...], 0.0))
    o_ref[...] = jnp.full((8, 128), ssum / cnt, dtype=jnp.float32)


def _mean_ea(edge_index, edge_attr):
    e = edge_attr.shape[0]
    rows = e // 128
    out = pl.pallas_call(
        _mean_ea_kernel,
        out_shape=jax.ShapeDtypeStruct((8, 128), jnp.float32),
    )(edge_index.reshape(2, rows, 128), edge_attr.reshape(rows, 128))
    return out[0, 0]


def _sc_edge_body(xl, xr, q0, q1, q2, q3, srcg, dstg2, eag, wvec,
                  attvec, zn, zd, nump, denp,
                  num_sh, den_sh, src_v, didx0, didx1, dst2_v, ea_v, ex_v,
                  rb0, rb1, qb0, qb1, db0, db1, w_v, att_v,
                  sa0, sa1, sq0, sq1, ss0, ss1, sd0, sd1):
    c = lax.axis_index("c")
    s = lax.axis_index("s")
    wid = c * NS + s
    qtabs = (q0, q1, q2, q3)
    rbufs = (rb0, rb1)
    qbufs = (qb0, qb1)
    dbufs = (db0, db1)
    didxs = (didx0, didx1)
    sas = (sa0, sa1)
    sqs = (sq0, sq1)
    sss = (ss0, ss1)
    sds = (sd0, sd1)

    pltpu.sync_copy(srcg.at[wid], src_v)
    pltpu.sync_copy(dstg2.at[wid], dst2_v)
    pltpu.sync_copy(eag.at[wid], ea_v)
    pltpu.sync_copy(wvec, w_v)
    pltpu.sync_copy(attvec, att_v)
    pltpu.sync_copy(zd.at[pl.ds(0, KA)], db0)
    pltpu.sync_copy(zd.at[pl.ds(0, KA)], db1)

    lanes_i = lax.iota(jnp.int32, LANES)
    nclamp = jnp.full((LANES,), N_NODES - 1, jnp.int32)

    def fill_didx(p, g):
        dch = dst2_v[g, pl.ds(0, LANES)]
        didxs[p][pl.ds(0, LANES)] = jnp.minimum(dch, nclamp)

    def issue_xl(p, g):
        pltpu.async_copy(xl.at[src_v.at[pl.ds(g * KA, KA)]], rbufs[p],
                         sas[p])

    def wait_xl(p, g):
        pltpu.make_async_copy(xl.at[src_v.at[pl.ds(g * KA, KA)]], rbufs[p],
                              sas[p]).wait()

    def issue_xr(p):
        pltpu.async_copy(xr.at[didxs[p]], rbufs[p], sas[p], add=True)

    def wait_xr(p):
        pltpu.make_async_copy(xr.at[didxs[p]], rbufs[p], sas[p]).wait()

    def compute_a(p, g):
        base = g * KA
        eav = ea_v[pl.ds(base, LANES)]
        eaus = [eav[u] for u in range(LANES)]
        accs = [jnp.zeros((LANES,), jnp.float32) for _ in range(LANES)]
        for cidx in range(D_H // LANES):
            wc = w_v[pl.ds(cidx * LANES, LANES)]
            ac = att_v[pl.ds(cidx * LANES, LANES)]
            for u in range(LANES):
                row = rbufs[p][u, pl.ds(cidx * LANES, LANES)]
                t = row + eaus[u] * wc
                t = jnp.maximum(t, 0.2 * t)
                accs[u] = accs[u] + t * ac
        e16 = jnp.zeros((LANES,), jnp.float32)
        for u in range(LANES):
            e16 = jnp.where(lanes_i == u, jnp.sum(accs[u]), e16)
        ex_v[pl.ds(base, LANES)] = jnp.exp(jnp.minimum(e16, 50.0))

    fill_didx(0, 0)
    issue_xl(0, 0)
    wait_xl(0, 0)
    issue_xr(0)
    fill_didx(1, 1)
    issue_xl(1, 1)

    def body_a(i, _):
        g0 = 2 * i
        g1 = g0 + 1
        wait_xl(1, g1)
        issue_xr(1)
        wait_xr(0)
        compute_a(0, g0)

        @pl.when(g0 + 2 < GA)
        def _():
            fill_didx(0, g0 + 2)
            issue_xl(0, g0 + 2)

        wait_xr(1)
        compute_a(1, g1)

        @pl.when(g1 + 2 < GA)
        def _():
            fill_didx(1, g1 + 2)
            issue_xl(1, g1 + 2)

        @pl.when(g0 + 2 < GA)
        def _():
            wait_xl(0, g0 + 2)
            issue_xr(0)

        return 0

    lax.fori_loop(0, GA // 2, body_a, 0)

    zeros_i = jnp.zeros((LANES,), jnp.int32)
    for q in range(NQ):
        pltpu.sync_copy(zn, num_sh.at[pl.ds(s * RPT, RPT)])
        if q == 0:
            pltpu.sync_copy(zd.at[pl.ds(0, RPT)],
                            den_sh.at[pl.ds(s * RPT, RPT)])
        plsc.subcore_barrier()

        def issue_qg(p, g):
            pltpu.async_copy(qtabs[q].at[src_v.at[pl.ds(g * KA, KA)]],
                             qbufs[p], sqs[p])

        def wait_qg(p, g):
            pltpu.make_async_copy(qtabs[q].at[src_v.at[pl.ds(g * KA, KA)]],
                                  qbufs[p], sqs[p]).wait()

        def scale(p, g):
            base = g * KA
            exch = ex_v[pl.ds(base, LANES)]
            for u in range(LANES):
                exj = exch[u]
                for cc in range(QW // LANES):
                    sl = pl.ds(cc * LANES, LANES)
                    qbufs[p][u, sl] = qbufs[p][u, sl] * exj
            if q == 0:
                plsc.store_scatter(dbufs[p], [lanes_i, zeros_i], exch)

        def issue_sc(p, g):
            pltpu.async_copy(qbufs[p], num_sh.at[dst2_v.at[g]], sss[p],
                             add=True)
            if q == 0:
                pltpu.async_copy(dbufs[p], den_sh.at[dst2_v.at[g]], sds[p],
                                 add=True)

        def wait_sc(p, g):
            pltpu.make_async_copy(qbufs[p], num_sh.at[dst2_v.at[g]],
                                  sss[p]).wait()
            if q == 0:
                pltpu.make_async_copy(dbufs[p], den_sh.at[dst2_v.at[g]],
                                      sds[p]).wait()

        issue_qg(0, 0)
        issue_qg(1, 1)

        def body_b(i, _):
            g0 = 2 * i
            g1 = g0 + 1
            wait_qg(0, g0)
            scale(0, g0)
            issue_sc(0, g0)
            wait_qg(1, g1)
            scale(1, g1)
            issue_sc(1, g1)

            @pl.when(g0 + 2 < GA)
            def _():
                wait_sc(0, g0)
                issue_qg(0, g0 + 2)

            @pl.when(g1 + 2 < GA)
            def _():
                wait_sc(1, g1)
                issue_qg(1, g1 + 2)

            return 0

        lax.fori_loop(0, GA // 2, body_b, 0)
        wait_sc(0, GA - 2)
        wait_sc(1, GA - 1)
        plsc.subcore_barrier()
        pltpu.sync_copy(num_sh.at[pl.ds(s * RPT, RPT)],
                        nump.at[c, q, pl.ds(s * RPT, RPT)])
        if q == 0:
            pltpu.sync_copy(den_sh.at[pl.ds(s * RPT, RPT)],
                            denp.at[c, pl.ds(s * RPT, RPT)])
        plsc.subcore_barrier()


def _sc_edge(xl, xr, quarters, srcg, dstg2, eag, wvec, attvec, zn, zd):
    mesh = plsc.VectorSubcoreMesh(core_axis_name="c", subcore_axis_name="s",
                                  num_cores=NC, num_subcores=NS)
    run = pl.kernel(
        _sc_edge_body,
        out_type=[
            jax.ShapeDtypeStruct((NC, NQ, NROWS, QW), jnp.float32),
            jax.ShapeDtypeStruct((NC, NROWS, DW), jnp.float32),
        ],
        mesh=mesh,
        compiler_params=pltpu.CompilerParams(needs_layout_passes=False,
                                             use_tc_tiling_on_sc=False),
        scratch_types=[
            pltpu.VMEM_SHARED((NROWS, QW), jnp.float32),
            pltpu.VMEM_SHARED((NROWS, DW), jnp.float32),
            pltpu.VMEM((EPT,), jnp.int32),
            pltpu.VMEM((KA,), jnp.int32),
            pltpu.VMEM((KA,), jnp.int32),
            pltpu.VMEM((GA, KA), jnp.int32),
            pltpu.VMEM((EPT,), jnp.float32),
            pltpu.VMEM((EPT,), jnp.float32),
            pltpu.VMEM((KA, D_H), jnp.float32),
            pltpu.VMEM((KA, D_H), jnp.float32),
            pltpu.VMEM((KA, QW), jnp.float32),
            pltpu.VMEM((KA, QW), jnp.float32),
            pltpu.VMEM((KA, DW), jnp.float32),
            pltpu.VMEM((KA, DW), jnp.float32),
            pltpu.VMEM((D_H,), jnp.float32),
            pltpu.VMEM((D_H,), jnp.float32),
            pltpu.SemaphoreType.DMA,
            pltpu.SemaphoreType.DMA,
            pltpu.SemaphoreType.DMA,
            pltpu.SemaphoreType.DMA,
            pltpu.SemaphoreType.DMA,
            pltpu.SemaphoreType.DMA,
            pltpu.SemaphoreType.DMA,
            pltpu.SemaphoreType.DMA,
        ],
    )
    return run(xl, xr, quarters[0], quarters[1], quarters[2], quarters[3],
               srcg, dstg2, eag, wvec, attvec, zn, zd)


def _div_kernel(nmp_ref, dnp_ref, b_ref, o_ref):
    num = nmp_ref[...]
    den = dnp_ref[...]
    ns = num[0] + num[1]
    d = den[0, :, 0] + den[1, :, 0]
    h = jnp.concatenate([ns[0], ns[1], ns[2], ns[3]], axis=1)
    h = h / (d[:, None] + 1e-16) + b_ref[...]
    o_ref[...] = jnp.where(h > 0, h, jnp.exp(h) - 1.0)


def _divide_elu(nump, denp, bias, block_rows=1000):
    return pl.pallas_call(
        _div_kernel,
        grid=(N_NODES // block_rows,),
        in_specs=[
            pl.BlockSpec((NC, NQ, block_rows, QW), lambda i: (0, 0, i, 0)),
            pl.BlockSpec((NC, block_rows, DW), lambda i: (0, i, 0)),
            pl.BlockSpec((1, D_H), lambda i: (0, 0)),
        ],
        out_specs=pl.BlockSpec((block_rows, D_H), lambda i: (i, 0)),
        out_shape=jax.ShapeDtypeStruct((N_NODES, D_H), jnp.float32),
    )(nump, denp, bias[None, :])


def _final_kernel(nmp_ref, dnp_ref, b_ref, wc_ref, bc_ref, o_ref, acc_ref):
    i = pl.program_id(0)
    num = nmp_ref[...]
    den = dnp_ref[...]
    ns = num[0] + num[1]
    d = den[0, :, 0] + den[1, :, 0]
    h = jnp.concatenate([ns[0], ns[1], ns[2], ns[3]], axis=1)
    h = h / (d[:, None] + 1e-16) + b_ref[...]
    h = jnp.where(h > 0, h, jnp.exp(h) - 1.0)
    part = jnp.sum(h.reshape(-1, 8, D_H), axis=0)

    @pl.when(i == 0)
    def _():
        acc_ref[...] = part

    @pl.when(i > 0)
    def _():
        acc_ref[...] = acc_ref[...] + part

    @pl.when(i == pl.num_programs(0) - 1)
    def _():
        pooled = jnp.sum(acc_ref[...], axis=0, keepdims=True) / N_NODES
        res = jnp.dot(pooled, wc_ref[...],
                      preferred_element_type=jnp.float32) + bc_ref[...]
        o_ref[...] = jnp.broadcast_to(res, (8, 128))


def _final(nump, denp, bias, wcTp, bcp, block_rows=1000):
    return pl.pallas_call(
        _final_kernel,
        grid=(N_NODES // block_rows,),
        in_specs=[
            pl.BlockSpec((NC, NQ, block_rows, QW), lambda i: (0, 0, i, 0)),
            pl.BlockSpec((NC, block_rows, DW), lambda i: (0, i, 0)),
            pl.BlockSpec((1, D_H), lambda i: (0, 0)),
            pl.BlockSpec((D_H, 128), lambda i: (0, 0)),
            pl.BlockSpec((1, 128), lambda i: (0, 0)),
        ],
        out_specs=pl.BlockSpec((8, 128), lambda i: (0, 0)),
        out_shape=jax.ShapeDtypeStruct((8, 128), jnp.float32),
        scratch_shapes=[pltpu.VMEM((8, D_H), jnp.float32)],
    )(nump, denp, bias[None, :], wcTp, bcp)


def kernel(x, edge_index, edge_attr, Wl1, bl1, Wr1, br1, We1, att1, bias1,
           Wl2, bl2, Wr2, br2, We2, att2, bias2, Wc, bc):
    n = x.shape[0]
    e = edge_attr.shape[0]
    src0 = edge_index[0].astype(jnp.int32)
    dst0 = edge_index[1].astype(jnp.int32)
    keep = src0 != dst0
    mean_ea = _mean_ea(edge_index.astype(jnp.int32), edge_attr)
    loop = jnp.arange(n, dtype=jnp.int32)
    pad = EP - e - n
    src_all = jnp.concatenate([src0, loop, jnp.zeros((pad,), jnp.int32)])
    dst_all = jnp.concatenate([
        jnp.where(keep, dst0, n), loop, jnp.full((pad,), n, jnp.int32)])
    ea_all = jnp.concatenate([
        edge_attr, jnp.full((n,), mean_ea, jnp.float32),
        jnp.zeros((pad,), jnp.float32)])

    srcg = src_all.reshape(NW, EPT)
    dstg2 = dst_all.reshape(NW, GA, KA)
    eag = ea_all.reshape(NW, EPT)
    zn = jnp.zeros((RPT, QW), jnp.float32)
    zd = jnp.zeros((RPT, DW), jnp.float32)

    def layer(xin, Wl, bl, Wr, br, We, att):
        wT = jnp.concatenate([Wl, Wr], axis=0).T
        bcat = jnp.concatenate([bl, br])
        y = _matmul(xin, wT, bcat)
        xlv = y[:, :D_H]
        xrv = y[:, D_H:]
        quarters = [xlv[:, i * QW:(i + 1) * QW] for i in range(NQ)]
        return _sc_edge(xlv, xrv, quarters, srcg, dstg2, eag,
                        We[:, 0], att, zn, zd)

    nump1, denp1 = layer(x, Wl1, bl1, Wr1, br1, We1, att1)
    h = _divide_elu(nump1, denp1, bias1)
    nump2, denp2 = layer(h, Wl2, bl2, Wr2, br2, We2, att2)
    wcTp = jnp.pad(Wc.T, ((0, 0), (0, 128 - Wc.shape[0])))
    bcp = jnp.pad(bc, (0, 128 - bc.shape[0]))[None, :]
    outp = _final(nump2, denp2, bias2, wcTp, bcp)
    return outp[0:1, 0:Wc.shape[0]]


# speedup vs baseline: 3.3260x; 1.0884x over previous
import jax
import jax.numpy as jnp
from jax import lax
from jax.experimental import pallas as pl
from jax.experimental.pallas import tpu as pltpu
from jax.experimental.pallas import tpu_sc as plsc

N_NODES = 10000
D_H = 512
NQ = 4
QW = D_H // NQ
NC, NS, LANES = 2, 16, 16
NW = NC * NS
EPT = 5376
EP = NW * EPT
KA = 16
GA = EPT // KA
KB = 32
GB = EPT // KB
NROWS = 10112
RPT = NROWS // NS
DW = 8


def _mm_kernel(x_ref, w_ref, b_ref, o_ref):
    o_ref[...] = (
        jnp.dot(x_ref[...], w_ref[...], preferred_element_type=jnp.float32)
        + b_ref[...]
    )


def _matmul(x, wT, b, block_rows=1000):
    n, k = x.shape
    m = wT.shape[1]
    return pl.pallas_call(
        _mm_kernel,
        grid=(n // block_rows,),
        in_specs=[
            pl.BlockSpec((block_rows, k), lambda i: (i, 0)),
            pl.BlockSpec((k, m), lambda i: (0, 0)),
            pl.BlockSpec((1, m), lambda i: (0, 0)),
        ],
        out_specs=pl.BlockSpec((block_rows, m), lambda i: (i, 0)),
        out_shape=jax.ShapeDtypeStruct((n, m), jnp.float32),
    )(x, wT, b[None, :])


def _mean_ea_kernel(ei_ref, ea_ref, o_ref):
    s_ids = ei_ref[0]
    d_ids = ei_ref[1]
    keep = s_ids != d_ids
    cnt = jnp.sum(keep.astype(jnp.float32))
    ssum = jnp.sum(jnp.where(keep, ea_ref[...], 0.0))
    o_ref[...] = jnp.full((8, 128), ssum / cnt, dtype=jnp.float32)


def _mean_ea(edge_index, edge_attr):
    e = edge_attr.shape[0]
    rows = e // 128
    out = pl.pallas_call(
        _mean_ea_kernel,
        out_shape=jax.ShapeDtypeStruct((8, 128), jnp.float32),
    )(edge_index.reshape(2, rows, 128), edge_attr.reshape(rows, 128))
    return out[0, 0]


def _sc_edge_body(xl, xr, q0, q1, q2, q3, srcg, dstg2, eag, wvec,
                  attvec, zn, zd, nump, denp,
                  num_sh, den_sh, src_v, didx0, didx1, dst2_v, ea_v, ex_v,
                  rb0, rb1, qb0, qb1, db0, db1, w_v, att_v,
                  sa0, sa1, sq0, sq1, ss0, ss1, sd0, sd1):
    c = lax.axis_index("c")
    s = lax.axis_index("s")
    wid = c * NS + s
    qtabs = (q0, q1, q2, q3)
    rbufs = (rb0, rb1)
    qbufs = (qb0, qb1)
    dbufs = (db0, db1)
    didxs = (didx0, didx1)
    sas = (sa0, sa1)
    sqs = (sq0, sq1)
    sss = (ss0, ss1)
    sds = (sd0, sd1)

    pltpu.sync_copy(srcg.at[wid], src_v)
    pltpu.sync_copy(dstg2.at[wid], dst2_v)
    pltpu.sync_copy(eag.at[wid], ea_v)
    pltpu.sync_copy(wvec, w_v)
    pltpu.sync_copy(attvec, att_v)
    pltpu.sync_copy(zd.at[pl.ds(0, KB)], db0)
    pltpu.sync_copy(zd.at[pl.ds(0, KB)], db1)

    lanes_i = lax.iota(jnp.int32, LANES)
    nclamp = jnp.full((LANES,), N_NODES - 1, jnp.int32)

    def fill_didx(p, row, half):
        dch = dst2_v[row, pl.ds(half * LANES, LANES)]
        didxs[p][pl.ds(0, LANES)] = jnp.minimum(dch, nclamp)

    def issue_xl(p, g):
        pltpu.async_copy(xl.at[src_v.at[pl.ds(g * KA, KA)]], rbufs[p],
                         sas[p])

    def wait_xl(p, g):
        pltpu.make_async_copy(xl.at[src_v.at[pl.ds(g * KA, KA)]], rbufs[p],
                              sas[p]).wait()

    def issue_xr(p):
        pltpu.async_copy(xr.at[didxs[p]], rbufs[p], sas[p], add=True)

    def wait_xr(p):
        pltpu.make_async_copy(xr.at[didxs[p]], rbufs[p], sas[p]).wait()

    def compute_a(p, i, par):
        base = (2 * i + par) * KA
        ea32 = ea_v[pl.ds(i * 2 * LANES, 2 * LANES)]
        eva, evb = plsc.unpack(ea32, format=plsc.PackFormat.INTERLEAVED,
                               preferred_element_type=jnp.float32)
        eaus = []
        for u in range(LANES):
            k = par * LANES + u
            eaus.append(eva[k // 2] if k % 2 == 0 else evb[k // 2])
        accs = [jnp.zeros((LANES,), jnp.float32) for _ in range(LANES)]
        for cidx in range(D_H // LANES):
            sl = pl.ds(cidx * LANES, LANES)
            wc = w_v[sl]
            ac = att_v[sl]
            for u in range(LANES):
                row = rbufs[p][u, sl]
                t = row + eaus[u] * wc
                t = jnp.maximum(t, 0.2 * t)
                accs[u] = accs[u] + t * ac
        e16 = jnp.zeros((LANES,), jnp.float32)
        for u in range(LANES):
            e16 = jnp.where(lanes_i == u, jnp.sum(accs[u]), e16)
        ex_v[pl.ds(base, LANES)] = jnp.exp(jnp.minimum(e16, 50.0))

    fill_didx(0, 0, 0)
    issue_xl(0, 0)
    wait_xl(0, 0)
    issue_xr(0)
    fill_didx(1, 0, 1)
    issue_xl(1, 1)

    def body_a(i, _):
        g0 = 2 * i
        g1 = g0 + 1
        wait_xl(1, g1)
        issue_xr(1)
        wait_xr(0)
        compute_a(0, i, 0)

        @pl.when(g0 + 2 < GA)
        def _():
            fill_didx(0, i + 1, 0)
            issue_xl(0, g0 + 2)

        wait_xr(1)
        compute_a(1, i, 1)

        @pl.when(g1 + 2 < GA)
        def _():
            fill_didx(1, i + 1, 1)
            issue_xl(1, g1 + 2)

        @pl.when(g0 + 2 < GA)
        def _():
            wait_xl(0, g0 + 2)
            issue_xr(0)

        return 0

    lax.fori_loop(0, GA // 2, body_a, 0)

    zeros_i = jnp.zeros((LANES,), jnp.int32)
    for q in range(NQ):
        pltpu.sync_copy(zn, num_sh.at[pl.ds(s * RPT, RPT)])
        if q == 0:
            pltpu.sync_copy(zd.at[pl.ds(0, RPT)],
                            den_sh.at[pl.ds(s * RPT, RPT)])
        plsc.subcore_barrier()

        def issue_qg(p, g):
            pltpu.async_copy(qtabs[q].at[src_v.at[pl.ds(g * KB, KB)]],
                             qbufs[p], sqs[p])

        def wait_qg(p, g):
            pltpu.make_async_copy(qtabs[q].at[src_v.at[pl.ds(g * KB, KB)]],
                                  qbufs[p], sqs[p]).wait()

        def scale(p, g):
            base = g * KB
            for sg in range(KB // LANES):
                exch = ex_v[pl.ds(base + sg * LANES, LANES)]
                for u in range(LANES):
                    exj = exch[u]
                    for cc in range(QW // LANES):
                        sl = pl.ds(cc * LANES, LANES)
                        qbufs[p][sg * LANES + u, sl] = (
                            qbufs[p][sg * LANES + u, sl] * exj)
                if q == 0:
                    plsc.store_scatter(
                        dbufs[p], [lanes_i + sg * LANES, zeros_i], exch)

        def issue_sc(p, g):
            pltpu.async_copy(qbufs[p], num_sh.at[dst2_v.at[g]], sss[p],
                             add=True)
            if q == 0:
                pltpu.async_copy(dbufs[p], den_sh.at[dst2_v.at[g]], sds[p],
                                 add=True)

        def wait_sc(p, g):
            pltpu.make_async_copy(qbufs[p], num_sh.at[dst2_v.at[g]],
                                  sss[p]).wait()
            if q == 0:
                pltpu.make_async_copy(dbufs[p], den_sh.at[dst2_v.at[g]],
                                      sds[p]).wait()

        issue_qg(0, 0)
        issue_qg(1, 1)

        def body_b(i, _):
            g0 = 2 * i
            g1 = g0 + 1
            wait_qg(0, g0)
            scale(0, g0)
            issue_sc(0, g0)
            wait_qg(1, g1)
            scale(1, g1)
            issue_sc(1, g1)

            @pl.when(g0 + 2 < GB)
            def _():
                wait_sc(0, g0)
                issue_qg(0, g0 + 2)

            @pl.when(g1 + 2 < GB)
            def _():
                wait_sc(1, g1)
                issue_qg(1, g1 + 2)

            return 0

        lax.fori_loop(0, GB // 2, body_b, 0)
        wait_sc(0, GB - 2)
        wait_sc(1, GB - 1)
        plsc.subcore_barrier()
        pltpu.sync_copy(num_sh.at[pl.ds(s * RPT, RPT)],
                        nump.at[c, q, pl.ds(s * RPT, RPT)])
        if q == 0:
            pltpu.sync_copy(den_sh.at[pl.ds(s * RPT, RPT)],
                            denp.at[c, pl.ds(s * RPT, RPT)])
        plsc.subcore_barrier()


def _sc_edge(xl, xr, quarters, srcg, dstg2, eag, wvec, attvec, zn, zd):
    mesh = plsc.VectorSubcoreMesh(core_axis_name="c", subcore_axis_name="s",
                                  num_cores=NC, num_subcores=NS)
    run = pl.kernel(
        _sc_edge_body,
        out_type=[
            jax.ShapeDtypeStruct((NC, NQ, NROWS, QW), jnp.float32),
            jax.ShapeDtypeStruct((NC, NROWS, DW), jnp.float32),
        ],
        mesh=mesh,
        compiler_params=pltpu.CompilerParams(needs_layout_passes=False,
                                             use_tc_tiling_on_sc=False),
        scratch_types=[
            pltpu.VMEM_SHARED((NROWS, QW), jnp.float32),
            pltpu.VMEM_SHARED((NROWS, DW), jnp.float32),
            pltpu.VMEM((EPT,), jnp.int32),
            pltpu.VMEM((KA,), jnp.int32),
            pltpu.VMEM((KA,), jnp.int32),
            pltpu.VMEM((GB, KB), jnp.int32),
            pltpu.VMEM((EPT,), jnp.bfloat16),
            pltpu.VMEM((EPT,), jnp.float32),
            pltpu.VMEM((KA, D_H), jnp.float32),
            pltpu.VMEM((KA, D_H), jnp.float32),
            pltpu.VMEM((KB, QW), jnp.float32),
            pltpu.VMEM((KB, QW), jnp.float32),
            pltpu.VMEM((KB, DW), jnp.float32),
            pltpu.VMEM((KB, DW), jnp.float32),
            pltpu.VMEM((D_H,), jnp.float32),
            pltpu.VMEM((D_H,), jnp.float32),
            pltpu.SemaphoreType.DMA,
            pltpu.SemaphoreType.DMA,
            pltpu.SemaphoreType.DMA,
            pltpu.SemaphoreType.DMA,
            pltpu.SemaphoreType.DMA,
            pltpu.SemaphoreType.DMA,
            pltpu.SemaphoreType.DMA,
            pltpu.SemaphoreType.DMA,
        ],
    )
    return run(xl, xr, quarters[0], quarters[1], quarters[2], quarters[3],
               srcg, dstg2, eag, wvec, attvec, zn, zd)


def _div_kernel(nmp_ref, dnp_ref, b_ref, o_ref):
    num = nmp_ref[...]
    den = dnp_ref[...]
    ns = num[0] + num[1]
    d = den[0, :, 0] + den[1, :, 0]
    h = jnp.concatenate([ns[0], ns[1], ns[2], ns[3]], axis=1)
    h = h / (d[:, None] + 1e-16) + b_ref[...]
    o_ref[...] = jnp.where(h > 0, h, jnp.exp(h) - 1.0)


def _divide_elu(nump, denp, bias, block_rows=1000):
    return pl.pallas_call(
        _div_kernel,
        grid=(N_NODES // block_rows,),
        in_specs=[
            pl.BlockSpec((NC, NQ, block_rows, QW), lambda i: (0, 0, i, 0)),
            pl.BlockSpec((NC, block_rows, DW), lambda i: (0, i, 0)),
            pl.BlockSpec((1, D_H), lambda i: (0, 0)),
        ],
        out_specs=pl.BlockSpec((block_rows, D_H), lambda i: (i, 0)),
        out_shape=jax.ShapeDtypeStruct((N_NODES, D_H), jnp.float32),
    )(nump, denp, bias[None, :])


def _final_kernel(nmp_ref, dnp_ref, b_ref, wc_ref, bc_ref, o_ref, acc_ref):
    i = pl.program_id(0)
    num = nmp_ref[...]
    den = dnp_ref[...]
    ns = num[0] + num[1]
    d = den[0, :, 0] + den[1, :, 0]
    h = jnp.concatenate([ns[0], ns[1], ns[2], ns[3]], axis=1)
    h = h / (d[:, None] + 1e-16) + b_ref[...]
    h = jnp.where(h > 0, h, jnp.exp(h) - 1.0)
    part = jnp.sum(h.reshape(-1, 8, D_H), axis=0)

    @pl.when(i == 0)
    def _():
        acc_ref[...] = part

    @pl.when(i > 0)
    def _():
        acc_ref[...] = acc_ref[...] + part

    @pl.when(i == pl.num_programs(0) - 1)
    def _():
        pooled = jnp.sum(acc_ref[...], axis=0, keepdims=True) / N_NODES
        res = jnp.dot(pooled, wc_ref[...],
                      preferred_element_type=jnp.float32) + bc_ref[...]
        o_ref[...] = jnp.broadcast_to(res, (8, 128))


def _final(nump, denp, bias, wcTp, bcp, block_rows=1000):
    return pl.pallas_call(
        _final_kernel,
        grid=(N_NODES // block_rows,),
        in_specs=[
            pl.BlockSpec((NC, NQ, block_rows, QW), lambda i: (0, 0, i, 0)),
            pl.BlockSpec((NC, block_rows, DW), lambda i: (0, i, 0)),
            pl.BlockSpec((1, D_H), lambda i: (0, 0)),
            pl.BlockSpec((D_H, 128), lambda i: (0, 0)),
            pl.BlockSpec((1, 128), lambda i: (0, 0)),
        ],
        out_specs=pl.BlockSpec((8, 128), lambda i: (0, 0)),
        out_shape=jax.ShapeDtypeStruct((8, 128), jnp.float32),
        scratch_shapes=[pltpu.VMEM((8, D_H), jnp.float32)],
    )(nump, denp, bias[None, :], wcTp, bcp)


def kernel(x, edge_index, edge_attr, Wl1, bl1, Wr1, br1, We1, att1, bias1,
           Wl2, bl2, Wr2, br2, We2, att2, bias2, Wc, bc):
    n = x.shape[0]
    e = edge_attr.shape[0]
    src0 = edge_index[0].astype(jnp.int32)
    dst0 = edge_index[1].astype(jnp.int32)
    keep = src0 != dst0
    mean_ea = _mean_ea(edge_index.astype(jnp.int32), edge_attr)
    loop = jnp.arange(n, dtype=jnp.int32)
    pad = EP - e - n
    src_all = jnp.concatenate([src0, loop, jnp.zeros((pad,), jnp.int32)])
    dst_all = jnp.concatenate([
        jnp.where(keep, dst0, n), loop, jnp.full((pad,), n, jnp.int32)])
    ea_all = jnp.concatenate([
        edge_attr, jnp.full((n,), mean_ea, jnp.float32),
        jnp.zeros((pad,), jnp.float32)])

    srcg = src_all.reshape(NW, EPT)
    dstg2 = dst_all.reshape(NW, GB, KB)
    eag = ea_all.astype(jnp.bfloat16).reshape(NW, EPT)
    zn = jnp.zeros((RPT, QW), jnp.float32)
    zd = jnp.zeros((RPT, DW), jnp.float32)

    def layer(xin, Wl, bl, Wr, br, We, att):
        wT = jnp.concatenate([Wl, Wr], axis=0).T
        bcat = jnp.concatenate([bl, br])
        y = _matmul(xin, wT, bcat)
        xlv = y[:, :D_H]
        xrv = y[:, D_H:]
        quarters = [xlv[:, i * QW:(i + 1) * QW] for i in range(NQ)]
        return _sc_edge(xlv, xrv, quarters, srcg, dstg2, eag,
                        We[:, 0], att, zn, zd)

    nump1, denp1 = layer(x, Wl1, bl1, Wr1, br1, We1, att1)
    h = _divide_elu(nump1, denp1, bias1)
    nump2, denp2 = layer(h, Wl2, bl2, Wr2, br2, We2, att2)
    wcTp = jnp.pad(Wc.T, ((0, 0), (0, 128 - Wc.shape[0])))
    bcp = jnp.pad(bc, (0, 128 - bc.shape[0]))[None, :]
    outp = _final(nump2, denp2, bias2, wcTp, bcp)
    return outp[0:1, 0:Wc.shape[0]]


# speedup vs baseline: 4.3403x; 1.3050x over previous
import jax
import jax.numpy as jnp
from jax import lax
from jax.experimental import pallas as pl
from jax.experimental.pallas import tpu as pltpu
from jax.experimental.pallas import tpu_sc as plsc

N_NODES = 10000
D_H = 512
NQ = 4
QW = D_H // NQ
NC, NS, LANES = 2, 16, 16
NW = NC * NS
EPT = 5376
EP = NW * EPT
KA = 16
GA = EPT // KA
KB = 32
GB = EPT // KB
NROWS = 10112
RPT = NROWS // NS
DW = 8


def _mm_kernel(x_ref, w_ref, b_ref, o_ref):
    o_ref[...] = (
        jnp.dot(x_ref[...], w_ref[...], preferred_element_type=jnp.float32)
        + b_ref[...]
    )


def _matmul(x, wT, b, block_rows=1000):
    n, k = x.shape
    m = wT.shape[1]
    return pl.pallas_call(
        _mm_kernel,
        grid=(n // block_rows,),
        in_specs=[
            pl.BlockSpec((block_rows, k), lambda i: (i, 0)),
            pl.BlockSpec((k, m), lambda i: (0, 0)),
            pl.BlockSpec((1, m), lambda i: (0, 0)),
        ],
        out_specs=pl.BlockSpec((block_rows, m), lambda i: (i, 0)),
        out_shape=jax.ShapeDtypeStruct((n, m), jnp.float32),
    )(x, wT, b[None, :])


def _mean_ea_kernel(ei_ref, ea_ref, o_ref):
    s_ids = ei_ref[0]
    d_ids = ei_ref[1]
    keep = s_ids != d_ids
    cnt = jnp.sum(keep.astype(jnp.float32))
    ssum = jnp.sum(jnp.where(keep, ea_ref[...], 0.0))
    o_ref[...] = jnp.full((8, 128), ssum / cnt, dtype=jnp.float32)


def _mean_ea(edge_index, edge_attr):
    e = edge_attr.shape[0]
    rows = e // 128
    out = pl.pallas_call(
        _mean_ea_kernel,
        out_shape=jax.ShapeDtypeStruct((8, 128), jnp.float32),
    )(edge_index.reshape(2, rows, 128), edge_attr.reshape(rows, 128))
    return out[0, 0]


def _sc_edge_body(xl, xr, q0, q1, q2, q3, srcg, dstg2, eag, wvec,
                  attvec, zn, zd, nump, denp,
                  num_sh, den_sh, src_v, didx0, didx1, dst2_v, ea_v, ex_v,
                  rl0, rl1, rr0, rr1, qb0, qb1, db0, db1, w_v, att_v,
                  sal0, sal1, sar0, sar1, sq0, sq1, ss0, ss1, sd0, sd1):
    c = lax.axis_index("c")
    s = lax.axis_index("s")
    wid = c * NS + s
    qtabs = (q0, q1, q2, q3)
    rlbufs = (rl0, rl1)
    rrbufs = (rr0, rr1)
    qbufs = (qb0, qb1)
    dbufs = (db0, db1)
    didxs = (didx0, didx1)
    sals = (sal0, sal1)
    sars = (sar0, sar1)
    sqs = (sq0, sq1)
    sss = (ss0, ss1)
    sds = (sd0, sd1)

    pltpu.sync_copy(srcg.at[wid], src_v)
    pltpu.sync_copy(dstg2.at[wid], dst2_v)
    pltpu.sync_copy(eag.at[wid], ea_v)
    pltpu.sync_copy(wvec, w_v)
    pltpu.sync_copy(attvec, att_v)
    pltpu.sync_copy(zd.at[pl.ds(0, KB)], db0)
    pltpu.sync_copy(zd.at[pl.ds(0, KB)], db1)

    lanes_i = lax.iota(jnp.int32, LANES)
    nclamp = jnp.full((LANES,), N_NODES - 1, jnp.int32)

    def fill_didx(p, row, half):
        dch = dst2_v[row, pl.ds(half * LANES, LANES)]
        didxs[p][pl.ds(0, LANES)] = jnp.minimum(dch, nclamp)

    def issue_ab(p, g):
        pltpu.async_copy(xl.at[src_v.at[pl.ds(g * KA, KA)]], rlbufs[p],
                         sals[p])
        pltpu.async_copy(xr.at[didxs[p]], rrbufs[p], sars[p])

    def wait_ab(p, g):
        pltpu.make_async_copy(xl.at[src_v.at[pl.ds(g * KA, KA)]], rlbufs[p],
                              sals[p]).wait()
        pltpu.make_async_copy(xr.at[didxs[p]], rrbufs[p], sars[p]).wait()

    def compute_a(p, i, par):
        base = (2 * i + par) * KA
        ea32 = ea_v[pl.ds(i * 2 * LANES, 2 * LANES)]
        eva, evb = plsc.unpack(ea32, format=plsc.PackFormat.INTERLEAVED,
                               preferred_element_type=jnp.float32)
        eaus = []
        for u in range(LANES):
            k = par * LANES + u
            eaus.append(eva[k // 2] if k % 2 == 0 else evb[k // 2])

        e16 = jnp.zeros((LANES,), jnp.float32)
        for ublk in range(2):
            u0 = ublk * (LANES // 2)

            def dot_chunk(c2, accs):
                sl32 = pl.ds(c2 * 2 * LANES, 2 * LANES)
                sl16 = pl.ds(c2 * LANES, LANES)
                sl16o = pl.ds(D_H // 2 + c2 * LANES, LANES)
                wce = w_v[sl16]
                wco = w_v[sl16o]
                ace = att_v[sl16]
                aco = att_v[sl16o]
                out = []
                for uu in range(LANES // 2):
                    u = u0 + uu
                    rle, rlo = plsc.unpack(
                        rlbufs[p][u, sl32],
                        format=plsc.PackFormat.INTERLEAVED,
                        preferred_element_type=jnp.float32)
                    rre, rro = plsc.unpack(
                        rrbufs[p][u, sl32],
                        format=plsc.PackFormat.INTERLEAVED,
                        preferred_element_type=jnp.float32)
                    t1 = rle + rre + eaus[u] * wce
                    t1 = jnp.maximum(t1, 0.2 * t1)
                    t2 = rlo + rro + eaus[u] * wco
                    t2 = jnp.maximum(t2, 0.2 * t2)
                    out.append(accs[uu] + t1 * ace + t2 * aco)
                return out

            accs = lax.fori_loop(
                0, D_H // (2 * LANES), dot_chunk,
                [jnp.zeros((LANES,), jnp.float32)
                 for _ in range(LANES // 2)])
            for uu in range(LANES // 2):
                e16 = jnp.where(lanes_i == u0 + uu, jnp.sum(accs[uu]), e16)
        ex_v[pl.ds(base, LANES)] = jnp.exp(jnp.minimum(e16, 50.0))

    fill_didx(0, 0, 0)
    issue_ab(0, 0)
    fill_didx(1, 0, 1)
    issue_ab(1, 1)

    def body_a(i, _):
        g0 = 2 * i
        g1 = g0 + 1
        wait_ab(0, g0)
        compute_a(0, i, 0)

        @pl.when(g0 + 2 < GA)
        def _():
            fill_didx(0, i + 1, 0)
            issue_ab(0, g0 + 2)

        wait_ab(1, g1)
        compute_a(1, i, 1)

        @pl.when(g1 + 2 < GA)
        def _():
            fill_didx(1, i + 1, 1)
            issue_ab(1, g1 + 2)

        return 0

    lax.fori_loop(0, GA // 2, body_a, 0)

    zeros_i = jnp.zeros((LANES,), jnp.int32)
    for q in range(NQ):
        pltpu.sync_copy(zn, num_sh.at[pl.ds(s * RPT, RPT)])
        if q == 0:
            pltpu.sync_copy(zd.at[pl.ds(0, RPT)],
                            den_sh.at[pl.ds(s * RPT, RPT)])
        plsc.subcore_barrier()

        def issue_qg(p, g):
            pltpu.async_copy(qtabs[q].at[src_v.at[pl.ds(g * KB, KB)]],
                             qbufs[p], sqs[p])

        def wait_qg(p, g):
            pltpu.make_async_copy(qtabs[q].at[src_v.at[pl.ds(g * KB, KB)]],
                                  qbufs[p], sqs[p]).wait()

        def scale(p, g):
            base = g * KB
            for sg in range(KB // LANES):
                exch = ex_v[pl.ds(base + sg * LANES, LANES)]
                for u in range(LANES):
                    exj = exch[u]
                    for cc in range(QW // LANES):
                        sl = pl.ds(cc * LANES, LANES)
                        qbufs[p][sg * LANES + u, sl] = (
                            qbufs[p][sg * LANES + u, sl] * exj)
                if q == 0:
                    plsc.store_scatter(
                        dbufs[p], [lanes_i + sg * LANES, zeros_i], exch)

        def issue_sc(p, g):
            pltpu.async_copy(qbufs[p], num_sh.at[dst2_v.at[g]], sss[p],
                             add=True)
            if q == 0:
                pltpu.async_copy(dbufs[p], den_sh.at[dst2_v.at[g]], sds[p],
                                 add=True)

        def wait_sc(p, g):
            pltpu.make_async_copy(qbufs[p], num_sh.at[dst2_v.at[g]],
                                  sss[p]).wait()
            if q == 0:
                pltpu.make_async_copy(dbufs[p], den_sh.at[dst2_v.at[g]],
                                      sds[p]).wait()

        issue_qg(0, 0)
        issue_qg(1, 1)

        def body_b(i, _):
            g0 = 2 * i
            g1 = g0 + 1
            wait_qg(0, g0)
            scale(0, g0)
            issue_sc(0, g0)
            wait_qg(1, g1)
            scale(1, g1)
            issue_sc(1, g1)

            @pl.when(g0 + 2 < GB)
            def _():
                wait_sc(0, g0)
                issue_qg(0, g0 + 2)

            @pl.when(g1 + 2 < GB)
            def _():
                wait_sc(1, g1)
                issue_qg(1, g1 + 2)

            return 0

        lax.fori_loop(0, GB // 2, body_b, 0)
        wait_sc(0, GB - 2)
        wait_sc(1, GB - 1)
        plsc.subcore_barrier()
        pltpu.sync_copy(num_sh.at[pl.ds(s * RPT, RPT)],
                        nump.at[c, q, pl.ds(s * RPT, RPT)])
        if q == 0:
            pltpu.sync_copy(den_sh.at[pl.ds(s * RPT, RPT)],
                            denp.at[c, pl.ds(s * RPT, RPT)])
        plsc.subcore_barrier()


def _sc_edge(xl, xr, quarters, srcg, dstg2, eag, wvec, attvec, zn, zd):
    mesh = plsc.VectorSubcoreMesh(core_axis_name="c", subcore_axis_name="s",
                                  num_cores=NC, num_subcores=NS)
    run = pl.kernel(
        _sc_edge_body,
        out_type=[
            jax.ShapeDtypeStruct((NC, NQ, NROWS, QW), jnp.float32),
            jax.ShapeDtypeStruct((NC, NROWS, DW), jnp.float32),
        ],
        mesh=mesh,
        compiler_params=pltpu.CompilerParams(needs_layout_passes=False,
                                             use_tc_tiling_on_sc=False),
        scratch_types=[
            pltpu.VMEM_SHARED((NROWS, QW), jnp.float32),
            pltpu.VMEM_SHARED((NROWS, DW), jnp.float32),
            pltpu.VMEM((EPT,), jnp.int32),
            pltpu.VMEM((KA,), jnp.int32),
            pltpu.VMEM((KA,), jnp.int32),
            pltpu.VMEM((GB, KB), jnp.int32),
            pltpu.VMEM((EPT,), jnp.bfloat16),
            pltpu.VMEM((EPT,), jnp.float32),
            pltpu.VMEM((KA, D_H), jnp.bfloat16),
            pltpu.VMEM((KA, D_H), jnp.bfloat16),
            pltpu.VMEM((KA, D_H), jnp.bfloat16),
            pltpu.VMEM((KA, D_H), jnp.bfloat16),
            pltpu.VMEM((KB, QW), jnp.float32),
            pltpu.VMEM((KB, QW), jnp.float32),
            pltpu.VMEM((KB, DW), jnp.float32),
            pltpu.VMEM((KB, DW), jnp.float32),
            pltpu.VMEM((D_H,), jnp.float32),
            pltpu.VMEM((D_H,), jnp.float32),
            pltpu.SemaphoreType.DMA,
            pltpu.SemaphoreType.DMA,
            pltpu.SemaphoreType.DMA,
            pltpu.SemaphoreType.DMA,
            pltpu.SemaphoreType.DMA,
            pltpu.SemaphoreType.DMA,
            pltpu.SemaphoreType.DMA,
            pltpu.SemaphoreType.DMA,
            pltpu.SemaphoreType.DMA,
            pltpu.SemaphoreType.DMA,
        ],
    )
    return run(xl, xr, quarters[0], quarters[1], quarters[2], quarters[3],
               srcg, dstg2, eag, wvec, attvec, zn, zd)


def _div_kernel(nmp_ref, dnp_ref, b_ref, o_ref):
    num = nmp_ref[...]
    den = dnp_ref[...]
    ns = num[0] + num[1]
    d = den[0, :, 0] + den[1, :, 0]
    h = jnp.concatenate([ns[0], ns[1], ns[2], ns[3]], axis=1)
    h = h / (d[:, None] + 1e-16) + b_ref[...]
    o_ref[...] = jnp.where(h > 0, h, jnp.exp(h) - 1.0)


def _divide_elu(nump, denp, bias, block_rows=1000):
    return pl.pallas_call(
        _div_kernel,
        grid=(N_NODES // block_rows,),
        in_specs=[
            pl.BlockSpec((NC, NQ, block_rows, QW), lambda i: (0, 0, i, 0)),
            pl.BlockSpec((NC, block_rows, DW), lambda i: (0, i, 0)),
            pl.BlockSpec((1, D_H), lambda i: (0, 0)),
        ],
        out_specs=pl.BlockSpec((block_rows, D_H), lambda i: (i, 0)),
        out_shape=jax.ShapeDtypeStruct((N_NODES, D_H), jnp.float32),
    )(nump, denp, bias[None, :])


def _final_kernel(nmp_ref, dnp_ref, b_ref, wc_ref, bc_ref, o_ref, acc_ref):
    i = pl.program_id(0)
    num = nmp_ref[...]
    den = dnp_ref[...]
    ns = num[0] + num[1]
    d = den[0, :, 0] + den[1, :, 0]
    h = jnp.concatenate([ns[0], ns[1], ns[2], ns[3]], axis=1)
    h = h / (d[:, None] + 1e-16) + b_ref[...]
    h = jnp.where(h > 0, h, jnp.exp(h) - 1.0)
    part = jnp.sum(h.reshape(-1, 8, D_H), axis=0)

    @pl.when(i == 0)
    def _():
        acc_ref[...] = part

    @pl.when(i > 0)
    def _():
        acc_ref[...] = acc_ref[...] + part

    @pl.when(i == pl.num_programs(0) - 1)
    def _():
        pooled = jnp.sum(acc_ref[...], axis=0, keepdims=True) / N_NODES
        res = jnp.dot(pooled, wc_ref[...],
                      preferred_element_type=jnp.float32) + bc_ref[...]
        o_ref[...] = jnp.broadcast_to(res, (8, 128))


def _final(nump, denp, bias, wcTp, bcp, block_rows=1000):
    return pl.pallas_call(
        _final_kernel,
        grid=(N_NODES // block_rows,),
        in_specs=[
            pl.BlockSpec((NC, NQ, block_rows, QW), lambda i: (0, 0, i, 0)),
            pl.BlockSpec((NC, block_rows, DW), lambda i: (0, i, 0)),
            pl.BlockSpec((1, D_H), lambda i: (0, 0)),
            pl.BlockSpec((D_H, 128), lambda i: (0, 0)),
            pl.BlockSpec((1, 128), lambda i: (0, 0)),
        ],
        out_specs=pl.BlockSpec((8, 128), lambda i: (0, 0)),
        out_shape=jax.ShapeDtypeStruct((8, 128), jnp.float32),
        scratch_shapes=[pltpu.VMEM((8, D_H), jnp.float32)],
    )(nump, denp, bias[None, :], wcTp, bcp)


def kernel(x, edge_index, edge_attr, Wl1, bl1, Wr1, br1, We1, att1, bias1,
           Wl2, bl2, Wr2, br2, We2, att2, bias2, Wc, bc):
    n = x.shape[0]
    e = edge_attr.shape[0]
    src0 = edge_index[0].astype(jnp.int32)
    dst0 = edge_index[1].astype(jnp.int32)
    keep = src0 != dst0
    mean_ea = _mean_ea(edge_index.astype(jnp.int32), edge_attr)
    loop = jnp.arange(n, dtype=jnp.int32)
    pad = EP - e - n
    src_all = jnp.concatenate([src0, loop, jnp.zeros((pad,), jnp.int32)])
    dst_all = jnp.concatenate([
        jnp.where(keep, dst0, n), loop, jnp.full((pad,), n, jnp.int32)])
    ea_all = jnp.concatenate([
        edge_attr, jnp.full((n,), mean_ea, jnp.float32),
        jnp.zeros((pad,), jnp.float32)])

    srcg = src_all.reshape(NW, EPT)
    dstg2 = dst_all.reshape(NW, GB, KB)
    eag = ea_all.astype(jnp.bfloat16).reshape(NW, EPT)
    zn = jnp.zeros((RPT, QW), jnp.float32)
    zd = jnp.zeros((RPT, DW), jnp.float32)

    def layer(xin, Wl, bl, Wr, br, We, att):
        wT = jnp.concatenate([Wl, Wr], axis=0).T
        bcat = jnp.concatenate([bl, br])
        y = _matmul(xin, wT, bcat)
        xlv = y[:, :D_H]
        xrv = y[:, D_H:]
        quarters = [xlv[:, i * QW:(i + 1) * QW] for i in range(NQ)]
        wde = jnp.concatenate([We[0::2, 0], We[1::2, 0]])
        attde = jnp.concatenate([att[0::2], att[1::2]])
        return _sc_edge(xlv.astype(jnp.bfloat16), xrv.astype(jnp.bfloat16),
                        quarters, srcg, dstg2, eag, wde, attde, zn, zd)

    nump1, denp1 = layer(x, Wl1, bl1, Wr1, br1, We1, att1)
    h = _divide_elu(nump1, denp1, bias1)
    nump2, denp2 = layer(h, Wl2, bl2, Wr2, br2, We2, att2)
    wcTp = jnp.pad(Wc.T, ((0, 0), (0, 128 - Wc.shape[0])))
    bcp = jnp.pad(bc, (0, 128 - bc.shape[0]))[None, :]
    outp = _final(nump2, denp2, bias2, wcTp, bcp)
    return outp[0:1, 0:Wc.shape[0]]


# speedup vs baseline: 4.3677x; 1.0063x over previous
import jax
import jax.numpy as jnp
from jax import lax
from jax.experimental import pallas as pl
from jax.experimental.pallas import tpu as pltpu
from jax.experimental.pallas import tpu_sc as plsc

N_NODES = 10000
D_H = 512
NQ = 4
QW = D_H // NQ
NC, NS, LANES = 2, 16, 16
NW = NC * NS
EPT = 5376
EP = NW * EPT
KA = 16
GA = EPT // KA
KB = 32
GB = EPT // KB
NROWS = 10112
RPT = NROWS // NS
DW = 8


def _emit_tables(res, xlb_ref, xrb_ref, q_refs):
    xlb_ref[...] = res[:, :D_H].astype(jnp.bfloat16)
    xrb_ref[...] = res[:, D_H:].astype(jnp.bfloat16)
    for k in range(NQ):
        q_refs[k][...] = res[:, k * QW:(k + 1) * QW]


def _table_outs(block_rows):
    out_shape = (
        [jax.ShapeDtypeStruct((N_NODES, D_H), jnp.bfloat16)] * 2
        + [jax.ShapeDtypeStruct((N_NODES, QW), jnp.float32)] * NQ)
    out_specs = (
        [pl.BlockSpec((block_rows, D_H), lambda i: (i, 0))] * 2
        + [pl.BlockSpec((block_rows, QW), lambda i: (i, 0))] * NQ)
    return out_shape, out_specs


def _mm_kernel(x_ref, w_ref, b_ref, xlb_ref, xrb_ref, *q_refs):
    res = (jnp.dot(x_ref[...], w_ref[...],
                   preferred_element_type=jnp.float32) + b_ref[...])
    _emit_tables(res, xlb_ref, xrb_ref, q_refs)


def _matmul(x, wT, b, block_rows=2000):
    n, k = x.shape
    m = wT.shape[1]
    out_shape, out_specs = _table_outs(block_rows)
    return pl.pallas_call(
        _mm_kernel,
        grid=(n // block_rows,),
        in_specs=[
            pl.BlockSpec((block_rows, k), lambda i: (i, 0)),
            pl.BlockSpec((k, m), lambda i: (0, 0)),
            pl.BlockSpec((1, m), lambda i: (0, 0)),
        ],
        out_specs=out_specs,
        out_shape=out_shape,
    )(x, wT, b[None, :])


def _mm_div_kernel(nmp_ref, dnp_ref, b1_ref, w_ref, b_ref,
                   xlb_ref, xrb_ref, *q_refs):
    num = nmp_ref[...]
    den = dnp_ref[...]
    ns = num[0] + num[1]
    d = den[0, :, 0] + den[1, :, 0]
    h = jnp.concatenate([ns[0], ns[1], ns[2], ns[3]], axis=1)
    h = h / (d[:, None] + 1e-16) + b1_ref[...]
    h = jnp.where(h > 0, h, jnp.exp(h) - 1.0)
    res = (jnp.dot(h, w_ref[...],
                   preferred_element_type=jnp.float32) + b_ref[...])
    _emit_tables(res, xlb_ref, xrb_ref, q_refs)


def _matmul_div(nump, denp, bias1, wT, b, block_rows=2000):
    m = wT.shape[1]
    out_shape, out_specs = _table_outs(block_rows)
    return pl.pallas_call(
        _mm_div_kernel,
        grid=(N_NODES // block_rows,),
        in_specs=[
            pl.BlockSpec((NC, NQ, block_rows, QW), lambda i: (0, 0, i, 0)),
            pl.BlockSpec((NC, block_rows, DW), lambda i: (0, i, 0)),
            pl.BlockSpec((1, D_H), lambda i: (0, 0)),
            pl.BlockSpec((D_H, m), lambda i: (0, 0)),
            pl.BlockSpec((1, m), lambda i: (0, 0)),
        ],
        out_specs=out_specs,
        out_shape=out_shape,
    )(nump, denp, bias1[None, :], wT, b[None, :])


def _mean_ea_kernel(ei_ref, ea_ref, o_ref):
    s_ids = ei_ref[0]
    d_ids = ei_ref[1]
    keep = s_ids != d_ids
    cnt = jnp.sum(keep.astype(jnp.float32))
    ssum = jnp.sum(jnp.where(keep, ea_ref[...], 0.0))
    o_ref[...] = jnp.full((8, 128), ssum / cnt, dtype=jnp.float32)


def _mean_ea(edge_index, edge_attr):
    e = edge_attr.shape[0]
    rows = e // 128
    out = pl.pallas_call(
        _mean_ea_kernel,
        out_shape=jax.ShapeDtypeStruct((8, 128), jnp.float32),
    )(edge_index.reshape(2, rows, 128), edge_attr.reshape(rows, 128))
    return out[0, 0]


def _sc_edge_body(xl, xr, q0, q1, q2, q3, srcg, dstg2, eag, wvec,
                  attvec, zn, zd, nump, denp,
                  num_sh, den_sh, src_v, didx0, didx1, dst2_v, ea_v, ex_v,
                  rl0, rl1, rr0, rr1, qb0, qb1, db0, db1, w_v, att_v,
                  sal0, sal1, sar0, sar1, sq0, sq1, ss0, ss1, sd0, sd1):
    c = lax.axis_index("c")
    s = lax.axis_index("s")
    wid = c * NS + s
    qtabs = (q0, q1, q2, q3)
    rlbufs = (rl0, rl1)
    rrbufs = (rr0, rr1)
    qbufs = (qb0, qb1)
    dbufs = (db0, db1)
    didxs = (didx0, didx1)
    sals = (sal0, sal1)
    sars = (sar0, sar1)
    sqs = (sq0, sq1)
    sss = (ss0, ss1)
    sds = (sd0, sd1)

    pltpu.sync_copy(srcg.at[wid], src_v)
    pltpu.sync_copy(dstg2.at[wid], dst2_v)
    pltpu.sync_copy(eag.at[wid], ea_v)
    pltpu.sync_copy(wvec, w_v)
    pltpu.sync_copy(attvec, att_v)
    pltpu.sync_copy(zd.at[pl.ds(0, KB)], db0)
    pltpu.sync_copy(zd.at[pl.ds(0, KB)], db1)

    lanes_i = lax.iota(jnp.int32, LANES)
    nclamp = jnp.full((LANES,), N_NODES - 1, jnp.int32)

    def fill_didx(p, row, half):
        dch = dst2_v[row, pl.ds(half * LANES, LANES)]
        didxs[p][pl.ds(0, LANES)] = jnp.minimum(dch, nclamp)

    def issue_ab(p, g):
        pltpu.async_copy(xl.at[src_v.at[pl.ds(g * KA, KA)]], rlbufs[p],
                         sals[p])
        pltpu.async_copy(xr.at[didxs[p]], rrbufs[p], sars[p])

    def wait_ab(p, g):
        pltpu.make_async_copy(xl.at[src_v.at[pl.ds(g * KA, KA)]], rlbufs[p],
                              sals[p]).wait()
        pltpu.make_async_copy(xr.at[didxs[p]], rrbufs[p], sars[p]).wait()

    def compute_a(p, i, par):
        base = (2 * i + par) * KA
        ea32 = ea_v[pl.ds(i * 2 * LANES, 2 * LANES)]
        eva, evb = plsc.unpack(ea32, format=plsc.PackFormat.INTERLEAVED,
                               preferred_element_type=jnp.float32)
        eaus = []
        for u in range(LANES):
            k = par * LANES + u
            eaus.append(eva[k // 2] if k % 2 == 0 else evb[k // 2])

        e16 = jnp.zeros((LANES,), jnp.float32)
        for ublk in range(2):
            u0 = ublk * (LANES // 2)

            def dot_chunk(c2, accs):
                sl32 = pl.ds(c2 * 2 * LANES, 2 * LANES)
                sl16 = pl.ds(c2 * LANES, LANES)
                sl16o = pl.ds(D_H // 2 + c2 * LANES, LANES)
                wce = w_v[sl16]
                wco = w_v[sl16o]
                ace = att_v[sl16]
                aco = att_v[sl16o]
                out = []
                for uu in range(LANES // 2):
                    u = u0 + uu
                    rle, rlo = plsc.unpack(
                        rlbufs[p][u, sl32],
                        format=plsc.PackFormat.INTERLEAVED,
                        preferred_element_type=jnp.float32)
                    rre, rro = plsc.unpack(
                        rrbufs[p][u, sl32],
                        format=plsc.PackFormat.INTERLEAVED,
                        preferred_element_type=jnp.float32)
                    t1 = rle + rre + eaus[u] * wce
                    t1 = jnp.maximum(t1, 0.2 * t1)
                    t2 = rlo + rro + eaus[u] * wco
                    t2 = jnp.maximum(t2, 0.2 * t2)
                    out.append(accs[uu] + t1 * ace + t2 * aco)
                return out

            accs = lax.fori_loop(
                0, D_H // (2 * LANES), dot_chunk,
                [jnp.zeros((LANES,), jnp.float32)
                 for _ in range(LANES // 2)])
            for uu in range(LANES // 2):
                e16 = jnp.where(lanes_i == u0 + uu, jnp.sum(accs[uu]), e16)
        ex_v[pl.ds(base, LANES)] = jnp.exp(jnp.minimum(e16, 50.0))

    fill_didx(0, 0, 0)
    issue_ab(0, 0)
    fill_didx(1, 0, 1)
    issue_ab(1, 1)

    def body_a(i, _):
        g0 = 2 * i
        g1 = g0 + 1
        wait_ab(0, g0)
        compute_a(0, i, 0)

        @pl.when(g0 + 2 < GA)
        def _():
            fill_didx(0, i + 1, 0)
            issue_ab(0, g0 + 2)

        wait_ab(1, g1)
        compute_a(1, i, 1)

        @pl.when(g1 + 2 < GA)
        def _():
            fill_didx(1, i + 1, 1)
            issue_ab(1, g1 + 2)

        return 0

    lax.fori_loop(0, GA // 2, body_a, 0)

    zeros_i = jnp.zeros((LANES,), jnp.int32)
    for q in range(NQ):
        pltpu.sync_copy(zn, num_sh.at[pl.ds(s * RPT, RPT)])
        if q == 0:
            pltpu.sync_copy(zd.at[pl.ds(0, RPT)],
                            den_sh.at[pl.ds(s * RPT, RPT)])
        plsc.subcore_barrier()

        def issue_qg(p, g):
            pltpu.async_copy(qtabs[q].at[src_v.at[pl.ds(g * KB, KB)]],
                             qbufs[p], sqs[p])

        def wait_qg(p, g):
            pltpu.make_async_copy(qtabs[q].at[src_v.at[pl.ds(g * KB, KB)]],
                                  qbufs[p], sqs[p]).wait()

        def scale(p, g):
            base = g * KB
            for sg in range(KB // LANES):
                exch = ex_v[pl.ds(base + sg * LANES, LANES)]
                for u in range(LANES):
                    exj = exch[u]
                    for cc in range(QW // LANES):
                        sl = pl.ds(cc * LANES, LANES)
                        qbufs[p][sg * LANES + u, sl] = (
                            qbufs[p][sg * LANES + u, sl] * exj)
                if q == 0:
                    plsc.store_scatter(
                        dbufs[p], [lanes_i + sg * LANES, zeros_i], exch)

        def issue_sc(p, g):
            pltpu.async_copy(qbufs[p], num_sh.at[dst2_v.at[g]], sss[p],
                             add=True)
            if q == 0:
                pltpu.async_copy(dbufs[p], den_sh.at[dst2_v.at[g]], sds[p],
                                 add=True)

        def wait_sc(p, g):
            pltpu.make_async_copy(qbufs[p], num_sh.at[dst2_v.at[g]],
                                  sss[p]).wait()
            if q == 0:
                pltpu.make_async_copy(dbufs[p], den_sh.at[dst2_v.at[g]],
                                      sds[p]).wait()

        issue_qg(0, 0)
        issue_qg(1, 1)

        def body_b(i, _):
            g0 = 2 * i
            g1 = g0 + 1
            wait_qg(0, g0)
            scale(0, g0)
            issue_sc(0, g0)
            wait_qg(1, g1)
            scale(1, g1)
            issue_sc(1, g1)

            @pl.when(g0 + 2 < GB)
            def _():
                wait_sc(0, g0)
                issue_qg(0, g0 + 2)

            @pl.when(g1 + 2 < GB)
            def _():
                wait_sc(1, g1)
                issue_qg(1, g1 + 2)

            return 0

        lax.fori_loop(0, GB // 2, body_b, 0)
        wait_sc(0, GB - 2)
        wait_sc(1, GB - 1)
        plsc.subcore_barrier()
        pltpu.sync_copy(num_sh.at[pl.ds(s * RPT, RPT)],
                        nump.at[c, q, pl.ds(s * RPT, RPT)])
        if q == 0:
            pltpu.sync_copy(den_sh.at[pl.ds(s * RPT, RPT)],
                            denp.at[c, pl.ds(s * RPT, RPT)])
        plsc.subcore_barrier()


def _sc_edge(xl, xr, quarters, srcg, dstg2, eag, wvec, attvec, zn, zd):
    mesh = plsc.VectorSubcoreMesh(core_axis_name="c", subcore_axis_name="s",
                                  num_cores=NC, num_subcores=NS)
    run = pl.kernel(
        _sc_edge_body,
        out_type=[
            jax.ShapeDtypeStruct((NC, NQ, NROWS, QW), jnp.float32),
            jax.ShapeDtypeStruct((NC, NROWS, DW), jnp.float32),
        ],
        mesh=mesh,
        compiler_params=pltpu.CompilerParams(needs_layout_passes=False,
                                             use_tc_tiling_on_sc=False),
        scratch_types=[
            pltpu.VMEM_SHARED((NROWS, QW), jnp.float32),
            pltpu.VMEM_SHARED((NROWS, DW), jnp.float32),
            pltpu.VMEM((EPT,), jnp.int32),
            pltpu.VMEM((KA,), jnp.int32),
            pltpu.VMEM((KA,), jnp.int32),
            pltpu.VMEM((GB, KB), jnp.int32),
            pltpu.VMEM((EPT,), jnp.bfloat16),
            pltpu.VMEM((EPT,), jnp.float32),
            pltpu.VMEM((KA, D_H), jnp.bfloat16),
            pltpu.VMEM((KA, D_H), jnp.bfloat16),
            pltpu.VMEM((KA, D_H), jnp.bfloat16),
            pltpu.VMEM((KA, D_H), jnp.bfloat16),
            pltpu.VMEM((KB, QW), jnp.float32),
            pltpu.VMEM((KB, QW), jnp.float32),
            pltpu.VMEM((KB, DW), jnp.float32),
            pltpu.VMEM((KB, DW), jnp.float32),
            pltpu.VMEM((D_H,), jnp.float32),
            pltpu.VMEM((D_H,), jnp.float32),
            pltpu.SemaphoreType.DMA,
            pltpu.SemaphoreType.DMA,
            pltpu.SemaphoreType.DMA,
            pltpu.SemaphoreType.DMA,
            pltpu.SemaphoreType.DMA,
            pltpu.SemaphoreType.DMA,
            pltpu.SemaphoreType.DMA,
            pltpu.SemaphoreType.DMA,
            pltpu.SemaphoreType.DMA,
            pltpu.SemaphoreType.DMA,
        ],
    )
    return run(xl, xr, quarters[0], quarters[1], quarters[2], quarters[3],
               srcg, dstg2, eag, wvec, attvec, zn, zd)


def _final_kernel(nmp_ref, dnp_ref, b_ref, wc_ref, bc_ref, o_ref, acc_ref):
    i = pl.program_id(0)
    num = nmp_ref[...]
    den = dnp_ref[...]
    ns = num[0] + num[1]
    d = den[0, :, 0] + den[1, :, 0]
    h = jnp.concatenate([ns[0], ns[1], ns[2], ns[3]], axis=1)
    h = h / (d[:, None] + 1e-16) + b_ref[...]
    h = jnp.where(h > 0, h, jnp.exp(h) - 1.0)
    part = jnp.sum(h.reshape(-1, 8, D_H), axis=0)

    @pl.when(i == 0)
    def _():
        acc_ref[...] = part

    @pl.when(i > 0)
    def _():
        acc_ref[...] = acc_ref[...] + part

    @pl.when(i == pl.num_programs(0) - 1)
    def _():
        pooled = jnp.sum(acc_ref[...], axis=0, keepdims=True) / N_NODES
        res = jnp.dot(pooled, wc_ref[...],
                      preferred_element_type=jnp.float32) + bc_ref[...]
        o_ref[...] = jnp.broadcast_to(res, (8, 128))


def _final(nump, denp, bias, wcTp, bcp, block_rows=1000):
    return pl.pallas_call(
        _final_kernel,
        grid=(N_NODES // block_rows,),
        in_specs=[
            pl.BlockSpec((NC, NQ, block_rows, QW), lambda i: (0, 0, i, 0)),
            pl.BlockSpec((NC, block_rows, DW), lambda i: (0, i, 0)),
            pl.BlockSpec((1, D_H), lambda i: (0, 0)),
            pl.BlockSpec((D_H, 128), lambda i: (0, 0)),
            pl.BlockSpec((1, 128), lambda i: (0, 0)),
        ],
        out_specs=pl.BlockSpec((8, 128), lambda i: (0, 0)),
        out_shape=jax.ShapeDtypeStruct((8, 128), jnp.float32),
        scratch_shapes=[pltpu.VMEM((8, D_H), jnp.float32)],
    )(nump, denp, bias[None, :], wcTp, bcp)


def kernel(x, edge_index, edge_attr, Wl1, bl1, Wr1, br1, We1, att1, bias1,
           Wl2, bl2, Wr2, br2, We2, att2, bias2, Wc, bc):
    n = x.shape[0]
    e = edge_attr.shape[0]
    src0 = edge_index[0].astype(jnp.int32)
    dst0 = edge_index[1].astype(jnp.int32)
    keep = src0 != dst0
    mean_ea = _mean_ea(edge_index.astype(jnp.int32), edge_attr)
    loop = jnp.arange(n, dtype=jnp.int32)
    pad = EP - e - n
    src_all = jnp.concatenate([src0, loop, jnp.zeros((pad,), jnp.int32)])
    dst_all = jnp.concatenate([
        jnp.where(keep, dst0, n), loop, jnp.full((pad,), n, jnp.int32)])
    ea_all = jnp.concatenate([
        edge_attr, jnp.full((n,), mean_ea, jnp.float32),
        jnp.zeros((pad,), jnp.float32)])

    srcg = src_all.reshape(NW, EPT)
    dstg2 = dst_all.reshape(NW, GB, KB)
    eag = ea_all.astype(jnp.bfloat16).reshape(NW, EPT)
    zn = jnp.zeros((RPT, QW), jnp.float32)
    zd = jnp.zeros((RPT, DW), jnp.float32)

    def sc_call(tabs, We, att):
        xlb, xrb = tabs[0], tabs[1]
        wde = jnp.concatenate([We[0::2, 0], We[1::2, 0]])
        attde = jnp.concatenate([att[0::2], att[1::2]])
        return _sc_edge(xlb, xrb, tabs[2:], srcg, dstg2, eag,
                        wde, attde, zn, zd)

    tabs1 = _matmul(x, jnp.concatenate([Wl1, Wr1], axis=0).T,
                    jnp.concatenate([bl1, br1]))
    nump1, denp1 = sc_call(tabs1, We1, att1)
    tabs2 = _matmul_div(nump1, denp1, bias1,
                        jnp.concatenate([Wl2, Wr2], axis=0).T,
                        jnp.concatenate([bl2, br2]))
    nump2, denp2 = sc_call(tabs2, We2, att2)
    wcTp = jnp.pad(Wc.T, ((0, 0), (0, 128 - Wc.shape[0])))
    bcp = jnp.pad(bc, (0, 128 - bc.shape[0]))[None, :]
    outp = _final(nump2, denp2, bias2, wcTp, bcp)
    return outp[0:1, 0:Wc.shape[0]]
